# Initial kernel scaffold; baseline (speedup 1.0000x reference)
import jax
import jax.numpy as jnp
from jax.experimental import pallas as pl


def kernel(e, a, edge_index, W_src, b_src, W_dst, b_dst, W_ang, b_ang, W_msg, b_msg, W1, b1, W2, b2, gamma, beta):
    raise NotImplementedError("write your pallas kernel here")


# speedup vs baseline: 2.2965x; 2.2965x over previous
import functools

import jax
import jax.numpy as jnp
from jax import lax
from jax.experimental import pallas as pl
from jax.experimental.pallas import tpu as pltpu
from jax.experimental.pallas import tpu_sc as plsc

N = 10000
E = 160000
D = 256
H = 128

NSUB = 16
E_PER_SUB = E // NSUB
EB = 80
BLOCKS = E_PER_SUB // EB
NP = 10240
ROWS_PER_SUB = NP // NSUB
ZROWS = 16

_f32 = jnp.float32

BN = 1000


def _nodeproj_body(e_ref, ws, bs, wd, bd, wm, bm, ps0, ps1, pd0, pd1, pm0, pm1):
    x = e_ref[...]
    ps = jnp.dot(x, ws[...], preferred_element_type=_f32) + bs[...]
    pd = jnp.dot(x, wd[...], preferred_element_type=_f32) + bd[...]
    pm = jnp.dot(x, wm[...], preferred_element_type=_f32) + bm[...]
    ps0[...] = ps[:, :H]
    ps1[...] = ps[:, H:]
    pd0[...] = pd[:, :H]
    pd1[...] = pd[:, H:]
    pm0[...] = pm[:, :H]
    pm1[...] = pm[:, H:]


def _node_proj(e, W_src, b_src, W_dst, b_dst, W_msg, b_msg):
    grid = (N // BN,)
    half = jax.ShapeDtypeStruct((N, H), _f32)
    wspec = pl.BlockSpec((D, D), lambda i: (0, 0))
    bspec = pl.BlockSpec((1, D), lambda i: (0, 0))
    return pl.pallas_call(
        _nodeproj_body,
        grid=grid,
        in_specs=[
            pl.BlockSpec((BN, D), lambda i: (i, 0)),
            wspec, bspec, wspec, bspec, wspec, bspec,
        ],
        out_specs=[pl.BlockSpec((BN, H), lambda i: (i, 0))] * 6,
        out_shape=[half] * 6,
    )(e, W_src, b_src, W_dst, b_dst, W_msg, b_msg)


BE = 640


def _angproj_body(a_ref, wa, ba, ang0, ang1):
    x = a_ref[...]
    ang = jnp.dot(x, wa[...], preferred_element_type=_f32) + ba[...]
    ang0[...] = ang[:, :H]
    ang1[...] = ang[:, H:]


def _ang_proj(a, W_ang, b_ang):
    grid = (E // BE,)
    half = jax.ShapeDtypeStruct((E, H), _f32)
    return pl.pallas_call(
        _angproj_body,
        grid=grid,
        in_specs=[
            pl.BlockSpec((BE, D), lambda i: (i, 0)),
            pl.BlockSpec((D, D), lambda i: (0, 0)),
            pl.BlockSpec((1, D), lambda i: (0, 0)),
        ],
        out_specs=[pl.BlockSpec((BE, H), lambda i: (i, 0))] * 2,
        out_shape=[half] * 2,
    )(a, W_ang, b_ang)


def _edge_body(ps0, ps1, pd0, pd1, pm0, pm1, ang0, ang1, srcs, dsts,
               out0, out1,
               src_v, dst_v, ps_v, pd_v, pm_v, ang_v, zbuf, agg_sh,
               sem0, sem1, sem2):
    c = lax.axis_index("c")
    s = lax.axis_index("s")

    zero = jnp.zeros((16,), _f32)

    def _zrow(i, carry):
        for j in range(8):
            zbuf[i, pl.ds(j * 16, 16)] = zero
        return carry

    lax.fori_loop(0, ZROWS, _zrow, 0)

    def _zcopy(k, carry):
        pltpu.sync_copy(zbuf, agg_sh.at[pl.ds(s * ROWS_PER_SUB + k * ZROWS, ZROWS)])
        return carry

    lax.fori_loop(0, ROWS_PER_SUB // ZROWS, _zcopy, 0)
    plsc.subcore_barrier()

    def _block(b, carry):
        e0 = s * E_PER_SUB + b * EB
        pltpu.sync_copy(srcs.at[pl.ds(e0, EB)], src_v)
        pltpu.sync_copy(dsts.at[pl.ds(e0, EB)], dst_v)

        @pl.when(c == 0)
        def _():
            cp0 = pltpu.async_copy(ps0.at[src_v], ps_v, sem0)
            cp1 = pltpu.async_copy(pd0.at[dst_v], pd_v, sem1)
            cp2 = pltpu.async_copy(pm0.at[src_v], pm_v, sem2)
            pltpu.sync_copy(ang0.at[pl.ds(e0, EB)], ang_v)
            cp0.wait()
            cp1.wait()
            cp2.wait()

        @pl.when(c == 1)
        def _():
            cp0 = pltpu.async_copy(ps1.at[src_v], ps_v, sem0)
            cp1 = pltpu.async_copy(pd1.at[dst_v], pd_v, sem1)
            cp2 = pltpu.async_copy(pm1.at[src_v], pm_v, sem2)
            pltpu.sync_copy(ang1.at[pl.ds(e0, EB)], ang_v)
            cp0.wait()
            cp1.wait()
            cp2.wait()

        def _row(i, rcarry):
            for j in range(8):
                sl = pl.ds(j * 16, 16)
                x = ps_v[i, sl] + pd_v[i, sl] + ang_v[i, sl]
                g = 1.0 / (1.0 + jnp.exp(-x))
                pm_v[i, sl] = g * pm_v[i, sl]
            return rcarry

        lax.fori_loop(0, EB, _row, 0)
        pltpu.sync_copy(pm_v, agg_sh.at[dst_v], add=True)
        return carry

    lax.fori_loop(0, BLOCKS, _block, 0)
    plsc.subcore_barrier()

    r0 = s * ROWS_PER_SUB

    @pl.when(c == 0)
    def _():
        pltpu.sync_copy(agg_sh.at[pl.ds(r0, ROWS_PER_SUB)],
                        out0.at[pl.ds(r0, ROWS_PER_SUB)])

    @pl.when(c == 1)
    def _():
        pltpu.sync_copy(agg_sh.at[pl.ds(r0, ROWS_PER_SUB)],
                        out1.at[pl.ds(r0, ROWS_PER_SUB)])


@functools.cache
def _edge_pass_fn():
  return pl.kernel(
    _edge_body,
    out_type=[jax.ShapeDtypeStruct((NP, H), _f32)] * 2,
    mesh=plsc.VectorSubcoreMesh(core_axis_name="c", subcore_axis_name="s"),
    scratch_types=[
        pltpu.VMEM((EB,), jnp.int32),
        pltpu.VMEM((EB,), jnp.int32),
        pltpu.VMEM((EB, H), _f32),
        pltpu.VMEM((EB, H), _f32),
        pltpu.VMEM((EB, H), _f32),
        pltpu.VMEM((EB, H), _f32),
        pltpu.VMEM((ZROWS, H), _f32),
        pltpu.VMEM_SHARED((NP, H), _f32),
        pltpu.SemaphoreType.DMA,
        pltpu.SemaphoreType.DMA,
        pltpu.SemaphoreType.DMA,
    ],
  )


BM = 1000


def _mlp_body(e_ref, a0_ref, a1_ref, w1, b1, w2, b2, gam, bet, out_ref):
    x = e_ref[...]
    w1v = w1[...]
    h = (jnp.dot(x, w1v[:D], preferred_element_type=_f32)
         + jnp.dot(a0_ref[...], w1v[D:D + H], preferred_element_type=_f32)
         + jnp.dot(a1_ref[...], w1v[D + H:], preferred_element_type=_f32)
         + b1[...])
    h = h * (1.0 / (1.0 + jnp.exp(-h)))
    h = jnp.dot(h, w2[...], preferred_element_type=_f32) + b2[...]
    xr = x + h
    mu = jnp.mean(xr, axis=-1, keepdims=True)
    xc = xr - mu
    var = jnp.mean(xc * xc, axis=-1, keepdims=True)
    out_ref[...] = xc * lax.rsqrt(var + 1e-5) * gam[...] + bet[...]


def _mlp_ln(e, agg0, agg1, W1, b1, W2, b2, gamma, beta):
    grid = (N // BM,)
    return pl.pallas_call(
        _mlp_body,
        grid=grid,
        in_specs=[
            pl.BlockSpec((BM, D), lambda i: (i, 0)),
            pl.BlockSpec((BM, H), lambda i: (i, 0)),
            pl.BlockSpec((BM, H), lambda i: (i, 0)),
            pl.BlockSpec((2 * D, D), lambda i: (0, 0)),
            pl.BlockSpec((1, D), lambda i: (0, 0)),
            pl.BlockSpec((D, D), lambda i: (0, 0)),
            pl.BlockSpec((1, D), lambda i: (0, 0)),
            pl.BlockSpec((1, D), lambda i: (0, 0)),
            pl.BlockSpec((1, D), lambda i: (0, 0)),
        ],
        out_specs=pl.BlockSpec((BM, D), lambda i: (i, 0)),
        out_shape=jax.ShapeDtypeStruct((N, D), _f32),
    )(e, agg0, agg1, W1, b1, W2, b2, gamma, beta)


def kernel(e, a, edge_index, W_src, b_src, W_dst, b_dst, W_ang, b_ang,
           W_msg, b_msg, W1, b1, W2, b2, gamma, beta):
    src = edge_index[0].astype(jnp.int32)
    dst = edge_index[1].astype(jnp.int32)
    ps0, ps1, pd0, pd1, pm0, pm1 = _node_proj(
        e, W_src, b_src.reshape(1, D), W_dst, b_dst.reshape(1, D),
        W_msg, b_msg.reshape(1, D))
    ang0, ang1 = _ang_proj(a, W_ang, b_ang.reshape(1, D))
    agg0, agg1 = _edge_pass_fn()(ps0, ps1, pd0, pd1, pm0, pm1, ang0, ang1,
                                 src, dst)
    return _mlp_ln(e, agg0, agg1, W1, b1.reshape(1, D), W2, b2.reshape(1, D),
                   gamma.reshape(1, D), beta.reshape(1, D))


# speedup vs baseline: 3.1387x; 1.3667x over previous
import functools

import jax
import jax.numpy as jnp
from jax import lax
from jax.experimental import pallas as pl
from jax.experimental.pallas import tpu as pltpu
from jax.experimental.pallas import tpu_sc as plsc

N = 10000
E = 160000
D = 256
H = 128

NSUB = 16
E_PER_SUB = E // NSUB
EB = 40
BLOCKS = E_PER_SUB // EB
NP = 10240
ROWS_PER_SUB = NP // NSUB
ZROWS = 16

_f32 = jnp.float32

BN = 1000


def _nodeproj_body(e_ref, ws, bs, wd, bd, wm, bm, ps0, ps1, pd0, pd1, pm0, pm1):
    x = e_ref[...]
    ps = jnp.dot(x, ws[...], preferred_element_type=_f32) + bs[...]
    pd = jnp.dot(x, wd[...], preferred_element_type=_f32) + bd[...]
    pm = jnp.dot(x, wm[...], preferred_element_type=_f32) + bm[...]
    ps0[...] = ps[:, :H]
    ps1[...] = ps[:, H:]
    pd0[...] = pd[:, :H]
    pd1[...] = pd[:, H:]
    pm0[...] = pm[:, :H]
    pm1[...] = pm[:, H:]


def _node_proj(e, W_src, b_src, W_dst, b_dst, W_msg, b_msg):
    grid = (N // BN,)
    half = jax.ShapeDtypeStruct((N, H), _f32)
    wspec = pl.BlockSpec((D, D), lambda i: (0, 0))
    bspec = pl.BlockSpec((1, D), lambda i: (0, 0))
    return pl.pallas_call(
        _nodeproj_body,
        grid=grid,
        in_specs=[
            pl.BlockSpec((BN, D), lambda i: (i, 0)),
            wspec, bspec, wspec, bspec, wspec, bspec,
        ],
        out_specs=[pl.BlockSpec((BN, H), lambda i: (i, 0))] * 6,
        out_shape=[half] * 6,
    )(e, W_src, b_src, W_dst, b_dst, W_msg, b_msg)


BE = 640


def _angproj_body(a_ref, wa, ba, ang0, ang1):
    x = a_ref[...]
    ang = jnp.dot(x, wa[...], preferred_element_type=_f32) + ba[...]
    ang0[...] = ang[:, :H]
    ang1[...] = ang[:, H:]


def _ang_proj(a, W_ang, b_ang):
    grid = (E // BE,)
    half = jax.ShapeDtypeStruct((E, H), _f32)
    return pl.pallas_call(
        _angproj_body,
        grid=grid,
        in_specs=[
            pl.BlockSpec((BE, D), lambda i: (i, 0)),
            pl.BlockSpec((D, D), lambda i: (0, 0)),
            pl.BlockSpec((1, D), lambda i: (0, 0)),
        ],
        out_specs=[pl.BlockSpec((BE, H), lambda i: (i, 0))] * 2,
        out_shape=[half] * 2,
    )(a, W_ang, b_ang)


def _edge_body(ps0, ps1, pd0, pd1, pm0, pm1, ang0, ang1, srcs, dsts,
               out0, out1,
               src_v0, src_v1, dst_v0, dst_v1,
               ps_v0, ps_v1, pd_v0, pd_v1, pm_v0, pm_v1, ang_v0, ang_v1,
               zbuf, agg_sh,
               sem_i0, sem_i1, sem_d0, sem_d1):
    c = lax.axis_index("c")
    s = lax.axis_index("s")
    srcv = (src_v0, src_v1)
    dstv = (dst_v0, dst_v1)
    psv = (ps_v0, ps_v1)
    pdv = (pd_v0, pd_v1)
    pmv = (pm_v0, pm_v1)
    angv = (ang_v0, ang_v1)
    sem_i = (sem_i0, sem_i1)
    sem_d = (sem_d0, sem_d1)

    zero = jnp.zeros((16,), _f32)

    def _zrow(i, carry):
        for j in range(8):
            zbuf[i, pl.ds(j * 16, 16)] = zero
        return carry

    lax.fori_loop(0, ZROWS, _zrow, 0)

    def _zcopy(k, carry):
        pltpu.sync_copy(zbuf, agg_sh.at[pl.ds(s * ROWS_PER_SUB + k * ZROWS, ZROWS)])
        return carry

    lax.fori_loop(0, ROWS_PER_SUB // ZROWS, _zcopy, 0)
    plsc.subcore_barrier()

    def e0_of(b):
        return s * E_PER_SUB + b * EB

    def start_idx(b, p):
        e0 = e0_of(b)
        pltpu.async_copy(srcs.at[pl.ds(e0, EB)], srcv[p], sem_i[p])
        pltpu.async_copy(dsts.at[pl.ds(e0, EB)], dstv[p], sem_i[p])

    def wait_idx(p):
        pltpu.make_async_copy(srcs.at[pl.ds(0, EB)], srcv[p], sem_i[p]).wait()
        pltpu.make_async_copy(dsts.at[pl.ds(0, EB)], dstv[p], sem_i[p]).wait()

    def start_gather(b, p):
        e0 = e0_of(b)

        @pl.when(c == 0)
        def _():
            pltpu.async_copy(ps0.at[srcv[p]], psv[p], sem_d[p])
            pltpu.async_copy(pd0.at[dstv[p]], pdv[p], sem_d[p])
            pltpu.async_copy(pm0.at[srcv[p]], pmv[p], sem_d[p])
            pltpu.async_copy(ang0.at[pl.ds(e0, EB)], angv[p], sem_d[p])

        @pl.when(c == 1)
        def _():
            pltpu.async_copy(ps1.at[srcv[p]], psv[p], sem_d[p])
            pltpu.async_copy(pd1.at[dstv[p]], pdv[p], sem_d[p])
            pltpu.async_copy(pm1.at[srcv[p]], pmv[p], sem_d[p])
            pltpu.async_copy(ang1.at[pl.ds(e0, EB)], angv[p], sem_d[p])

    def wait_gather(p):
        pltpu.make_async_copy(ps0.at[srcv[p]], psv[p], sem_d[p]).wait()
        pltpu.make_async_copy(pd0.at[dstv[p]], pdv[p], sem_d[p]).wait()
        pltpu.make_async_copy(pm0.at[srcv[p]], pmv[p], sem_d[p]).wait()
        pltpu.make_async_copy(ang0.at[pl.ds(0, EB)], angv[p], sem_d[p]).wait()

    def compute(p):
        ps_v, pd_v, pm_v, ang_v = psv[p], pdv[p], pmv[p], angv[p]

        def _row(i, rcarry):
            for j in range(8):
                sl = pl.ds(j * 16, 16)
                x = ps_v[i, sl] + pd_v[i, sl] + ang_v[i, sl]
                g = 1.0 / (1.0 + jnp.exp(-x))
                pm_v[i, sl] = g * pm_v[i, sl]
            return rcarry

        lax.fori_loop(0, EB, _row, 0)

    start_idx(0, 0)
    wait_idx(0)
    start_gather(0, 0)
    start_idx(1, 1)

    def _body(b, p):
        @pl.when(b + 1 < BLOCKS)
        def _():
            wait_idx(1 - p)
            start_gather(b + 1, 1 - p)

        wait_gather(p)
        compute(p)
        pltpu.sync_copy(pmv[p], agg_sh.at[dstv[p]], add=True)

        @pl.when(b + 2 < BLOCKS)
        def _():
            start_idx(b + 2, p)

    def _pair(t, carry):
        _body(2 * t, 0)
        _body(2 * t + 1, 1)
        return carry

    lax.fori_loop(0, BLOCKS // 2, _pair, 0)
    plsc.subcore_barrier()

    r0 = s * ROWS_PER_SUB

    @pl.when(c == 0)
    def _():
        pltpu.sync_copy(agg_sh.at[pl.ds(r0, ROWS_PER_SUB)],
                        out0.at[pl.ds(r0, ROWS_PER_SUB)])

    @pl.when(c == 1)
    def _():
        pltpu.sync_copy(agg_sh.at[pl.ds(r0, ROWS_PER_SUB)],
                        out1.at[pl.ds(r0, ROWS_PER_SUB)])


@functools.cache
def _edge_pass_fn():
  return pl.kernel(
    _edge_body,
    out_type=[jax.ShapeDtypeStruct((NP, H), _f32)] * 2,
    mesh=plsc.VectorSubcoreMesh(core_axis_name="c", subcore_axis_name="s"),
    scratch_types=(
        [pltpu.VMEM((EB,), jnp.int32)] * 4
        + [pltpu.VMEM((EB, H), _f32)] * 8
        + [pltpu.VMEM((ZROWS, H), _f32),
           pltpu.VMEM_SHARED((NP, H), _f32),
           pltpu.SemaphoreType.DMA,
           pltpu.SemaphoreType.DMA,
           pltpu.SemaphoreType.DMA,
           pltpu.SemaphoreType.DMA]
    ),
  )


BM = 1000


def _mlp_body(e_ref, a0_ref, a1_ref, w1, b1, w2, b2, gam, bet, out_ref):
    x = e_ref[...]
    w1v = w1[...]
    h = (jnp.dot(x, w1v[:D], preferred_element_type=_f32)
         + jnp.dot(a0_ref[...], w1v[D:D + H], preferred_element_type=_f32)
         + jnp.dot(a1_ref[...], w1v[D + H:], preferred_element_type=_f32)
         + b1[...])
    h = h * (1.0 / (1.0 + jnp.exp(-h)))
    h = jnp.dot(h, w2[...], preferred_element_type=_f32) + b2[...]
    xr = x + h
    mu = jnp.mean(xr, axis=-1, keepdims=True)
    xc = xr - mu
    var = jnp.mean(xc * xc, axis=-1, keepdims=True)
    out_ref[...] = xc * lax.rsqrt(var + 1e-5) * gam[...] + bet[...]


def _mlp_ln(e, agg0, agg1, W1, b1, W2, b2, gamma, beta):
    grid = (N // BM,)
    return pl.pallas_call(
        _mlp_body,
        grid=grid,
        in_specs=[
            pl.BlockSpec((BM, D), lambda i: (i, 0)),
            pl.BlockSpec((BM, H), lambda i: (i, 0)),
            pl.BlockSpec((BM, H), lambda i: (i, 0)),
            pl.BlockSpec((2 * D, D), lambda i: (0, 0)),
            pl.BlockSpec((1, D), lambda i: (0, 0)),
            pl.BlockSpec((D, D), lambda i: (0, 0)),
            pl.BlockSpec((1, D), lambda i: (0, 0)),
            pl.BlockSpec((1, D), lambda i: (0, 0)),
            pl.BlockSpec((1, D), lambda i: (0, 0)),
        ],
        out_specs=pl.BlockSpec((BM, D), lambda i: (i, 0)),
        out_shape=jax.ShapeDtypeStruct((N, D), _f32),
    )(e, agg0, agg1, W1, b1, W2, b2, gamma, beta)


def kernel(e, a, edge_index, W_src, b_src, W_dst, b_dst, W_ang, b_ang,
           W_msg, b_msg, W1, b1, W2, b2, gamma, beta):
    src = edge_index[0].astype(jnp.int32)
    dst = edge_index[1].astype(jnp.int32)
    ps0, ps1, pd0, pd1, pm0, pm1 = _node_proj(
        e, W_src, b_src.reshape(1, D), W_dst, b_dst.reshape(1, D),
        W_msg, b_msg.reshape(1, D))
    ang0, ang1 = _ang_proj(a, W_ang, b_ang.reshape(1, D))
    agg0, agg1 = _edge_pass_fn()(ps0, ps1, pd0, pd1, pm0, pm1, ang0, ang1,
                                 src, dst)
    return _mlp_ln(e, agg0, agg1, W1, b1.reshape(1, D), W2, b2.reshape(1, D),
                   gamma.reshape(1, D), beta.reshape(1, D))
